# Initial kernel scaffold; baseline (speedup 1.0000x reference)
import jax
import jax.numpy as jnp
from jax.experimental import pallas as pl


def kernel(x, edge_index, W1, b1, W2, b2):
    raise NotImplementedError("write your pallas kernel here")


# speedup vs baseline: 14.3572x; 14.3572x over previous
import functools

import jax
import jax.numpy as jnp
from jax import lax
from jax.experimental import pallas as pl
from jax.experimental.pallas import tpu as pltpu
from jax.experimental.pallas import tpu_sc as plsc

NC = 2
NS = 16
NW = NC * NS
K = 80
BLK = 2000


def _sc_mesh():
    return plsc.VectorSubcoreMesh(
        core_axis_name="c", subcore_axis_name="s", num_cores=NC, num_subcores=NS
    )


def _make_prop_kernel(npad, d, nwin):
    rpt = npad // NS

    @functools.partial(
        pl.kernel,
        out_type=jax.ShapeDtypeStruct((NC, npad, d), jnp.float32),
        mesh=_sc_mesh(),
        scratch_types=[
            pltpu.VMEM((nwin, K), jnp.int32),
            pltpu.VMEM((nwin, K), jnp.int32),
            pltpu.VMEM((K, d), jnp.float32),
            pltpu.VMEM_SHARED((npad, d), jnp.float32),
            pltpu.SemaphoreType.DMA,
        ],
    )
    def prop_kernel(
        g_hbm, src_hbm, dst_hbm, zeros_hbm, out_hbm, src_v, dst_v, rows_v, acc_sh, sem
    ):
        c = lax.axis_index("c")
        s = lax.axis_index("s")
        wid = s * NC + c
        pltpu.sync_copy(src_hbm.at[wid], src_v)
        pltpu.sync_copy(dst_hbm.at[wid], dst_v)
        pltpu.sync_copy(
            zeros_hbm.at[pl.ds(s * rpt, rpt)], acc_sh.at[pl.ds(s * rpt, rpt)]
        )
        plsc.subcore_barrier()

        @pl.loop(0, nwin)
        def _(w):
            pltpu.async_copy(g_hbm.at[src_v.at[w]], rows_v, sem).wait()
            pltpu.sync_copy(rows_v, acc_sh.at[dst_v.at[w]], add=True)

        plsc.subcore_barrier()
        pltpu.sync_copy(
            acc_sh.at[pl.ds(s * rpt, rpt)], out_hbm.at[c, pl.ds(s * rpt, rpt)]
        )

    return prop_kernel


def _dinv_from_deg(deg_ref):
    deg = deg_ref[0, :, 0:1] + deg_ref[1, :, 0:1] + 1.0
    return lax.rsqrt(deg)


def _deg_spec():
    return pl.BlockSpec((2, BLK, 128), lambda i: (0, i, 0))


def _tc1_body(deg_ref, x_ref, w_ref, h_ref, g_ref):
    dinv = _dinv_from_deg(deg_ref)
    h = jnp.dot(x_ref[...], w_ref[...], preferred_element_type=jnp.float32)
    h_ref[...] = h
    g_ref[...] = h * dinv


def _tc2_body(deg_ref, s_ref, h_ref, w_ref, b_ref, h2_ref, g2_ref):
    dinv = _dinv_from_deg(deg_ref)
    stot = s_ref[0] + s_ref[1]
    out1 = jnp.maximum(stot * dinv + h_ref[...] * (dinv * dinv) + b_ref[...], 0.0)
    h2 = jnp.dot(out1, w_ref[...], preferred_element_type=jnp.float32)
    h2_ref[...] = h2
    g2_ref[...] = h2 * dinv


def _tc3_body(deg_ref, s_ref, h_ref, b_ref, o_ref):
    dinv = _dinv_from_deg(deg_ref)
    stot = s_ref[0] + s_ref[1]
    o_ref[...] = jnp.maximum(
        stot * dinv + h_ref[...] * (dinv * dinv) + b_ref[...], 0.0
    )


def _row_spec(d):
    return pl.BlockSpec((BLK, d), lambda i: (i, 0))


def _pair_spec(d):
    return pl.BlockSpec((2, BLK, d), lambda i: (0, i, 0))


def _full_spec(r, c):
    return pl.BlockSpec((r, c), lambda i: (0, 0))


def kernel(x, edge_index, W1, b1, W2, b2):
    n, d = x.shape
    e = edge_index.shape[1]
    nwin = e // K // NW
    grid = (n // BLK,)
    npad = ((n // NS + 7) // 8 * 8) * NS

    src2 = edge_index[0].reshape(NW, nwin, K)
    dst2 = edge_index[1].reshape(NW, nwin, K)
    zeros_d = jnp.zeros((npad, d), jnp.float32)
    ones_nd = jnp.ones((n, d), jnp.float32)
    b1r = b1.reshape(1, d)
    b2r = b2.reshape(1, d)

    prop = _make_prop_kernel(npad, d, nwin)
    deg16 = prop(ones_nd, src2, dst2, zeros_d)

    h1, g1 = pl.pallas_call(
        _tc1_body,
        grid=grid,
        in_specs=[_deg_spec(), _row_spec(d), _full_spec(d, d)],
        out_specs=[_row_spec(d), _row_spec(d)],
        out_shape=[
            jax.ShapeDtypeStruct((n, d), jnp.float32),
            jax.ShapeDtypeStruct((n, d), jnp.float32),
        ],
    )(deg16, x, W1)

    s1 = prop(g1, src2, dst2, zeros_d)

    h2, g2 = pl.pallas_call(
        _tc2_body,
        grid=grid,
        in_specs=[
            _deg_spec(),
            _pair_spec(d),
            _row_spec(d),
            _full_spec(d, d),
            _full_spec(1, d),
        ],
        out_specs=[_row_spec(d), _row_spec(d)],
        out_shape=[
            jax.ShapeDtypeStruct((n, d), jnp.float32),
            jax.ShapeDtypeStruct((n, d), jnp.float32),
        ],
    )(deg16, s1, h1, W2, b1r)

    s2 = prop(g2, src2, dst2, zeros_d)

    out = pl.pallas_call(
        _tc3_body,
        grid=grid,
        in_specs=[_deg_spec(), _pair_spec(d), _row_spec(d), _full_spec(1, d)],
        out_specs=_row_spec(d),
        out_shape=jax.ShapeDtypeStruct((n, d), jnp.float32),
    )(deg16, s2, h2, b2r)

    return out
